# Initial kernel scaffold; baseline (speedup 1.0000x reference)
import jax
import jax.numpy as jnp
from jax.experimental import pallas as pl


def kernel(piece_indices, piece_values, weight, bias):
    raise NotImplementedError("write your pallas kernel here")


# speedup vs baseline: 5.0714x; 5.0714x over previous
import functools

import jax
import jax.numpy as jnp
from jax import lax
from jax.experimental import pallas as pl
from jax.experimental.pallas import tpu as pltpu
from jax.experimental.pallas import tpu_sc as plsc

NUM_CORES = 2
NUM_SUBCORES = 16
NUM_WORKERS = NUM_CORES * NUM_SUBCORES
LANES = 16

CHUNK_ROWS = 16


def _sc_encode(idx_flat, val_flat, weight, bias, *, N, K, D):
    rows_per_w = N // NUM_WORKERS
    lk_per_w = rows_per_w * K
    chunk_lk = CHUNK_ROWS * K
    n_chunks = rows_per_w // CHUNK_ROWS
    d_vecs = D // LANES

    mesh = plsc.VectorSubcoreMesh(
        core_axis_name="c", subcore_axis_name="s",
        num_cores=NUM_CORES, num_subcores=NUM_SUBCORES)

    @functools.partial(
        pl.kernel,
        out_type=jax.ShapeDtypeStruct((N, D), jnp.float32),
        mesh=mesh,
        scratch_types=[
            pltpu.VMEM((lk_per_w,), jnp.int32),
            pltpu.VMEM((lk_per_w,), jnp.float32),
            pltpu.VMEM((chunk_lk, D), jnp.float32),
            pltpu.VMEM((CHUNK_ROWS, D), jnp.float32),
            pltpu.VMEM((D,), jnp.float32),
            pltpu.SemaphoreType.DMA,
        ],
    )
    def sc_kernel(idx_hbm, val_hbm, w_hbm, bias_hbm, out_hbm,
                  idx_v, val_v, rows_v, out_v, bias_v, sem):
        wid = lax.axis_index("s") * NUM_CORES + lax.axis_index("c")
        base_lk = wid * lk_per_w
        base_row = wid * rows_per_w

        pltpu.sync_copy(idx_hbm.at[pl.ds(base_lk, lk_per_w)], idx_v)
        pltpu.sync_copy(val_hbm.at[pl.ds(base_lk, lk_per_w)], val_v)
        pltpu.sync_copy(bias_hbm, bias_v)

        bias_regs = [bias_v[pl.ds(LANES * j, LANES)] for j in range(d_vecs)]

        @pl.loop(0, n_chunks)
        def chunk_body(c):
            lk0 = c * chunk_lk
            pltpu.async_copy(
                w_hbm.at[idx_v.at[pl.ds(lk0, chunk_lk)]], rows_v, sem).wait()
            for r in range(CHUNK_ROWS):
                acc = list(bias_regs)
                vv = val_v[pl.ds(lk0 + (r // 2) * LANES, LANES)]
                for k in range(K):
                    lr = r * K + k
                    lane = jnp.full((LANES,), (lr % LANES), jnp.int32)
                    v = jnp.take_along_axis(vv, lane, axis=0,
                                            mode="promise_in_bounds")
                    for j in range(d_vecs):
                        w_vec = rows_v[lr, pl.ds(LANES * j, LANES)]
                        acc[j] = acc[j] + w_vec * v
                for j in range(d_vecs):
                    out_v[r, pl.ds(LANES * j, LANES)] = acc[j]
            pltpu.sync_copy(
                out_v, out_hbm.at[pl.ds(base_row + c * CHUNK_ROWS, CHUNK_ROWS), :])

    return sc_kernel(idx_flat, val_flat, weight, bias)


def kernel(piece_indices, piece_values, weight, bias):
    B, S, K = piece_indices.shape
    D = weight.shape[1]
    N = B * S
    idx_flat = piece_indices.reshape(N * K)
    val_flat = piece_values.reshape(N * K)
    out = _sc_encode(idx_flat, val_flat, weight, bias, N=N, K=K, D=D)
    return out.reshape(B, S, D)


# speedup vs baseline: 6.4731x; 1.2764x over previous
import functools

import jax
import jax.numpy as jnp
from jax import lax
from jax.experimental import pallas as pl
from jax.experimental.pallas import tpu as pltpu
from jax.experimental.pallas import tpu_sc as plsc

NUM_CORES = 2
NUM_SUBCORES = 16
NUM_WORKERS = NUM_CORES * NUM_SUBCORES
LANES = 16

CHUNK_ROWS = 8


def _sc_encode(idx_flat, val_flat, weight, bias, *, N, K, D):
    rows_per_w = N // NUM_WORKERS
    lk_per_w = rows_per_w * K
    chunk_lk = CHUNK_ROWS * K
    n_chunks = rows_per_w // CHUNK_ROWS
    d_vecs = D // LANES

    mesh = plsc.VectorSubcoreMesh(
        core_axis_name="c", subcore_axis_name="s",
        num_cores=NUM_CORES, num_subcores=NUM_SUBCORES)

    @functools.partial(
        pl.kernel,
        out_type=jax.ShapeDtypeStruct((N, D), jnp.float32),
        mesh=mesh,
        scratch_types=[
            pltpu.VMEM((lk_per_w,), jnp.int32),
            pltpu.VMEM((lk_per_w,), jnp.float32),
            pltpu.VMEM((chunk_lk, D), jnp.float32),
            pltpu.VMEM((chunk_lk, D), jnp.float32),
            pltpu.VMEM((CHUNK_ROWS, D), jnp.float32),
            pltpu.VMEM((CHUNK_ROWS, D), jnp.float32),
            pltpu.VMEM((D,), jnp.float32),
            pltpu.SemaphoreType.DMA,
            pltpu.SemaphoreType.DMA,
            pltpu.SemaphoreType.DMA,
            pltpu.SemaphoreType.DMA,
        ],
    )
    def sc_kernel(idx_hbm, val_hbm, w_hbm, bias_hbm, out_hbm,
                  idx_v, val_v, rows_a, rows_b, out_a, out_b, bias_v,
                  gsem_a, gsem_b, ssem_a, ssem_b):
        wid = lax.axis_index("s") * NUM_CORES + lax.axis_index("c")
        base_lk = wid * lk_per_w
        base_row = wid * rows_per_w

        pltpu.sync_copy(idx_hbm.at[pl.ds(base_lk, lk_per_w)], idx_v)
        pltpu.sync_copy(val_hbm.at[pl.ds(base_lk, lk_per_w)], val_v)
        pltpu.sync_copy(bias_hbm, bias_v)

        bias_regs = [bias_v[pl.ds(LANES * j, LANES)] for j in range(d_vecs)]

        def gather(c, rows, sem):
            pltpu.async_copy(
                w_hbm.at[idx_v.at[pl.ds(c * chunk_lk, chunk_lk)]], rows, sem)

        def gather_wait(c, rows, sem):
            pltpu.make_async_copy(
                w_hbm.at[idx_v.at[pl.ds(c * chunk_lk, chunk_lk)]], rows,
                sem).wait()

        def out_slice(c):
            return out_hbm.at[pl.ds(base_row + c * CHUNK_ROWS, CHUNK_ROWS), :]

        def compute(c, rows_v, out_v):
            lk0 = c * chunk_lk
            for r in range(CHUNK_ROWS):
                acc = list(bias_regs)
                vv = val_v[pl.ds(lk0 + (r // 2) * LANES, LANES)]
                for k in range(K):
                    lr = r * K + k
                    lane = jnp.full((LANES,), (lr % LANES), jnp.int32)
                    v = jnp.take_along_axis(vv, lane, axis=0,
                                            mode="promise_in_bounds")
                    for j in range(d_vecs):
                        w_vec = rows_v[lr, pl.ds(LANES * j, LANES)]
                        acc[j] = acc[j] + w_vec * v
                for j in range(d_vecs):
                    out_v[r, pl.ds(LANES * j, LANES)] = acc[j]

        gather(0, rows_a, gsem_a)

        @pl.loop(0, n_chunks, step=2)
        def chunk_pair(c):
            gather(c + 1, rows_b, gsem_b)
            gather_wait(c, rows_a, gsem_a)

            @pl.when(c >= 2)
            def _():
                pltpu.make_async_copy(out_a, out_slice(c - 2), ssem_a).wait()

            compute(c, rows_a, out_a)
            pltpu.async_copy(out_a, out_slice(c), ssem_a)

            @pl.when(c + 2 < n_chunks)
            def _():
                gather(c + 2, rows_a, gsem_a)

            gather_wait(c + 1, rows_b, gsem_b)

            @pl.when(c >= 2)
            def _():
                pltpu.make_async_copy(out_b, out_slice(c - 1), ssem_b).wait()

            compute(c + 1, rows_b, out_b)
            pltpu.async_copy(out_b, out_slice(c + 1), ssem_b)

        pltpu.make_async_copy(out_a, out_slice(n_chunks - 2), ssem_a).wait()
        pltpu.make_async_copy(out_b, out_slice(n_chunks - 1), ssem_b).wait()

    return sc_kernel(idx_flat, val_flat, weight, bias)


def kernel(piece_indices, piece_values, weight, bias):
    B, S, K = piece_indices.shape
    D = weight.shape[1]
    N = B * S
    idx_flat = piece_indices.reshape(N * K)
    val_flat = piece_values.reshape(N * K)
    out = _sc_encode(idx_flat, val_flat, weight, bias, N=N, K=K, D=D)
    return out.reshape(B, S, D)


# speedup vs baseline: 7.4869x; 1.1566x over previous
import functools

import jax
import jax.numpy as jnp
from jax import lax
from jax.experimental import pallas as pl
from jax.experimental.pallas import tpu as pltpu
from jax.experimental.pallas import tpu_sc as plsc

NUM_CORES = 2
NUM_SUBCORES = 16
NUM_WORKERS = NUM_CORES * NUM_SUBCORES
LANES = 16

CHUNK_ROWS = 8


def _sc_encode(idx_flat, val_flat, w_bf16, bias, *, N, K, D):
    rows_per_w = N // NUM_WORKERS
    lk_per_w = rows_per_w * K
    chunk_lk = CHUNK_ROWS * K
    n_chunks = rows_per_w // CHUNK_ROWS
    d_vecs = D // LANES
    p_vecs = D // (2 * LANES)

    mesh = plsc.VectorSubcoreMesh(
        core_axis_name="c", subcore_axis_name="s",
        num_cores=NUM_CORES, num_subcores=NUM_SUBCORES)

    @functools.partial(
        pl.kernel,
        out_type=jax.ShapeDtypeStruct((N, D), jnp.float32),
        mesh=mesh,
        compiler_params=pltpu.CompilerParams(
            needs_layout_passes=False, use_tc_tiling_on_sc=False),
        scratch_types=[
            pltpu.VMEM((lk_per_w,), jnp.int32),
            pltpu.VMEM((lk_per_w,), jnp.float32),
            pltpu.VMEM((chunk_lk, D), jnp.bfloat16),
            pltpu.VMEM((chunk_lk, D), jnp.bfloat16),
            pltpu.VMEM((CHUNK_ROWS, D), jnp.float32),
            pltpu.VMEM((CHUNK_ROWS, D), jnp.float32),
            pltpu.VMEM((D,), jnp.float32),
            pltpu.SemaphoreType.DMA,
            pltpu.SemaphoreType.DMA,
            pltpu.SemaphoreType.DMA,
            pltpu.SemaphoreType.DMA,
        ],
    )
    def sc_kernel(idx_hbm, val_hbm, w_hbm, bias_hbm, out_hbm,
                  idx_v, val_v, rows_a, rows_b, out_a, out_b, bias_v,
                  gsem_a, gsem_b, ssem_a, ssem_b):
        wid = lax.axis_index("s") * NUM_CORES + lax.axis_index("c")
        base_lk = wid * lk_per_w
        base_row = wid * rows_per_w

        pltpu.sync_copy(idx_hbm.at[pl.ds(base_lk, lk_per_w)], idx_v)
        pltpu.sync_copy(val_hbm.at[pl.ds(base_lk, lk_per_w)], val_v)
        pltpu.sync_copy(bias_hbm, bias_v)

        bias_regs = [
            plsc.pack(bias_v[pl.ds(2 * LANES * j, LANES)],
                      bias_v[pl.ds(2 * LANES * j + LANES, LANES)],
                      format=plsc.PackFormat.INTERLEAVED)
            for j in range(p_vecs)
        ]

        def gather(c, rows, sem):
            pltpu.async_copy(
                w_hbm.at[idx_v.at[pl.ds(c * chunk_lk, chunk_lk)]], rows, sem)

        def gather_wait(c, rows, sem):
            pltpu.make_async_copy(
                w_hbm.at[idx_v.at[pl.ds(c * chunk_lk, chunk_lk)]], rows,
                sem).wait()

        def out_slice(c):
            return out_hbm.at[pl.ds(base_row + c * CHUNK_ROWS, CHUNK_ROWS), :]

        def compute(c, rows_v, out_v):
            lk0 = c * chunk_lk
            for r in range(CHUNK_ROWS):
                acc = list(bias_regs)
                vv = val_v[pl.ds(lk0 + (r // 2) * LANES, LANES)]
                for k in range(K):
                    lr = r * K + k
                    lane = jnp.full((LANES,), (lr % LANES), jnp.int32)
                    v = jnp.take_along_axis(vv, lane, axis=0,
                                            mode="promise_in_bounds")
                    vb = plsc.pack(v, v, format=plsc.PackFormat.INTERLEAVED)
                    for j in range(p_vecs):
                        wv = rows_v[lr, pl.ds(2 * LANES * j, 2 * LANES)]
                        acc[j] = acc[j] + wv * vb
                for j in range(p_vecs):
                    a_f32, b_f32 = plsc.unpack(
                        acc[j], format=plsc.PackFormat.INTERLEAVED)
                    out_v[r, pl.ds(2 * LANES * j, LANES)] = a_f32
                    out_v[r, pl.ds(2 * LANES * j + LANES, LANES)] = b_f32

        gather(0, rows_a, gsem_a)

        @pl.loop(0, n_chunks, step=2)
        def chunk_pair(c):
            gather(c + 1, rows_b, gsem_b)
            gather_wait(c, rows_a, gsem_a)

            @pl.when(c >= 2)
            def _():
                pltpu.make_async_copy(out_a, out_slice(c - 2), ssem_a).wait()

            compute(c, rows_a, out_a)
            pltpu.async_copy(out_a, out_slice(c), ssem_a)

            @pl.when(c + 2 < n_chunks)
            def _():
                gather(c + 2, rows_a, gsem_a)

            gather_wait(c + 1, rows_b, gsem_b)

            @pl.when(c >= 2)
            def _():
                pltpu.make_async_copy(out_b, out_slice(c - 1), ssem_b).wait()

            compute(c + 1, rows_b, out_b)
            pltpu.async_copy(out_b, out_slice(c + 1), ssem_b)

        pltpu.make_async_copy(out_a, out_slice(n_chunks - 2), ssem_a).wait()
        pltpu.make_async_copy(out_b, out_slice(n_chunks - 1), ssem_b).wait()

    return sc_kernel(idx_flat, val_flat, w_bf16, bias)


def _prep_weight(weight):
    V, D = weight.shape
    wb = weight.astype(jnp.bfloat16).reshape(V, D // 32, 2, 16)
    return wb.transpose(0, 1, 3, 2).reshape(V, D)


def kernel(piece_indices, piece_values, weight, bias):
    B, S, K = piece_indices.shape
    D = weight.shape[1]
    N = B * S
    idx_flat = piece_indices.reshape(N * K)
    val_flat = piece_values.reshape(N * K)
    w_bf16 = _prep_weight(weight)
    out = _sc_encode(idx_flat, val_flat, w_bf16, bias, N=N, K=K, D=D)
    return out.reshape(B, S, D)


# speedup vs baseline: 13.8709x; 1.8527x over previous
import functools

import jax
import jax.numpy as jnp
from jax import lax
from jax.experimental import pallas as pl
from jax.experimental.pallas import tpu as pltpu
from jax.experimental.pallas import tpu_sc as plsc

NUM_CORES = 2
NUM_SUBCORES = 16
NUM_WORKERS = NUM_CORES * NUM_SUBCORES
LANES = 16

CHUNK_ROWS = 16


def _sc_encode(idx2d, val2d, w_bf16, bias, *, B, S, K, D):
    N = B * S
    SK = S * K
    rows_per_w = N // NUM_WORKERS
    b_per_w = B // NUM_WORKERS
    lk_per_w = rows_per_w * K
    chunk_lk = CHUNK_ROWS * K
    n_chunks = rows_per_w // CHUNK_ROWS
    chunks_per_b = S // CHUNK_ROWS
    p_vecs = D // (2 * LANES)

    mesh = plsc.VectorSubcoreMesh(
        core_axis_name="c", subcore_axis_name="s",
        num_cores=NUM_CORES, num_subcores=NUM_SUBCORES)

    @functools.partial(
        pl.kernel,
        out_type=jax.ShapeDtypeStruct((B, S, D), jnp.float32),
        mesh=mesh,
        compiler_params=pltpu.CompilerParams(
            needs_layout_passes=False, use_tc_tiling_on_sc=False),
        scratch_types=[
            pltpu.VMEM((b_per_w, SK), jnp.int32),
            pltpu.VMEM((b_per_w, SK), jnp.float32),
            pltpu.VMEM((chunk_lk, D), jnp.bfloat16),
            pltpu.VMEM((chunk_lk, D), jnp.bfloat16),
            pltpu.VMEM((CHUNK_ROWS, D), jnp.float32),
            pltpu.VMEM((CHUNK_ROWS, D), jnp.float32),
            pltpu.VMEM((D,), jnp.float32),
            pltpu.SemaphoreType.DMA,
            pltpu.SemaphoreType.DMA,
            pltpu.SemaphoreType.DMA,
            pltpu.SemaphoreType.DMA,
        ],
    )
    def sc_kernel(idx_hbm, val_hbm, w_hbm, bias_hbm, out_hbm,
                  idx_v, val_v, rows_a, rows_b, out_a, out_b, bias_v,
                  gsem_a, gsem_b, ssem_a, ssem_b):
        wid = lax.axis_index("s") * NUM_CORES + lax.axis_index("c")
        base_b = wid * b_per_w

        pltpu.sync_copy(idx_hbm.at[pl.ds(base_b, b_per_w), :], idx_v)
        pltpu.sync_copy(val_hbm.at[pl.ds(base_b, b_per_w), :], val_v)
        pltpu.sync_copy(bias_hbm, bias_v)

        bias_regs = [
            plsc.pack(bias_v[pl.ds(2 * LANES * j, LANES)],
                      bias_v[pl.ds(2 * LANES * j + LANES, LANES)],
                      format=plsc.PackFormat.INTERLEAVED)
            for j in range(p_vecs)
        ]

        def chunk_pos(c):
            return c // chunks_per_b, (c % chunks_per_b) * chunk_lk

        def gather(c, rows, sem):
            br, col = chunk_pos(c)
            pltpu.async_copy(
                w_hbm.at[idx_v.at[br, pl.ds(col, chunk_lk)]], rows, sem)

        def gather_wait(c, rows, sem):
            br, col = chunk_pos(c)
            pltpu.make_async_copy(
                w_hbm.at[idx_v.at[br, pl.ds(col, chunk_lk)]], rows,
                sem).wait()

        def out_slice(c):
            return out_hbm.at[base_b + c // chunks_per_b,
                              pl.ds((c % chunks_per_b) * CHUNK_ROWS,
                                    CHUNK_ROWS), :]

        def compute(c, rows_v, out_v):
            br, col = chunk_pos(c)
            for r in range(CHUNK_ROWS):
                vv = val_v[br, pl.ds(col + (r // 2) * LANES, LANES)]
                vbs = []
                for k in range(K):
                    lane = jnp.full((LANES,), (r % 2) * K + k, jnp.int32)
                    v = jnp.take_along_axis(vv, lane, axis=0,
                                            mode="promise_in_bounds")
                    vbs.append(
                        plsc.pack(v, v, format=plsc.PackFormat.INTERLEAVED))
                for j in range(p_vecs):
                    p = [rows_v[r * K + k, pl.ds(2 * LANES * j, 2 * LANES)]
                         * vbs[k] for k in range(K)]
                    acc = (((p[0] + p[1]) + (p[2] + p[3]))
                           + ((p[4] + p[5]) + (p[6] + p[7]))) + bias_regs[j]
                    a_f32, b_f32 = plsc.unpack(
                        acc, format=plsc.PackFormat.INTERLEAVED)
                    out_v[r, pl.ds(2 * LANES * j, LANES)] = a_f32
                    out_v[r, pl.ds(2 * LANES * j + LANES, LANES)] = b_f32

        gather(0, rows_a, gsem_a)

        @pl.loop(0, n_chunks, step=2)
        def chunk_pair(c):
            gather(c + 1, rows_b, gsem_b)
            gather_wait(c, rows_a, gsem_a)

            @pl.when(c >= 2)
            def _():
                pltpu.make_async_copy(out_a, out_slice(c - 2), ssem_a).wait()

            compute(c, rows_a, out_a)
            pltpu.async_copy(out_a, out_slice(c), ssem_a)

            @pl.when(c + 2 < n_chunks)
            def _():
                gather(c + 2, rows_a, gsem_a)

            gather_wait(c + 1, rows_b, gsem_b)

            @pl.when(c >= 2)
            def _():
                pltpu.make_async_copy(out_b, out_slice(c - 1), ssem_b).wait()

            compute(c + 1, rows_b, out_b)
            pltpu.async_copy(out_b, out_slice(c + 1), ssem_b)

        pltpu.make_async_copy(out_a, out_slice(n_chunks - 2), ssem_a).wait()
        pltpu.make_async_copy(out_b, out_slice(n_chunks - 1), ssem_b).wait()

    return sc_kernel(idx2d, val2d, w_bf16, bias)


def _prep_weight(weight):
    V, D = weight.shape
    wb = weight.astype(jnp.bfloat16).reshape(V, D // 32, 2, 16)
    return wb.transpose(0, 1, 3, 2).reshape(V, D)


def kernel(piece_indices, piece_values, weight, bias):
    B, S, K = piece_indices.shape
    D = weight.shape[1]
    idx2d = piece_indices.reshape(B, S * K)
    val2d = piece_values.reshape(B, S * K)
    w_bf16 = _prep_weight(weight)
    return _sc_encode(idx2d, val2d, w_bf16, bias, B=B, S=S, K=K, D=D)
